# Initial kernel scaffold; baseline (speedup 1.0000x reference)
import jax
import jax.numpy as jnp
from jax.experimental import pallas as pl


def kernel(x):
    raise NotImplementedError("write your pallas kernel here")


# speedup vs baseline: 1.0783x; 1.0783x over previous
import jax
import jax.numpy as jnp
import numpy as np
from jax import lax
from jax.experimental import pallas as pl
from jax.experimental.pallas import tpu as pltpu

ROWS = 64
COLS = 1_000_000
BC = 8192
CW = 1024
NCHUNK = BC // CW
GRID = (COLS + BC - 1) // BC

_KS0 = np.uint32(0)
_KS1 = np.uint32(42)
_KS2 = np.uint32(0 ^ 42 ^ 0x1BD11BDA)
_ROT0 = (13, 15, 26, 6)
_ROT1 = (17, 29, 16, 24)


def _rounds(x0, x1, rots):
    for r in rots:
        x0 = x0 + x1
        x1 = (x1 << r) | (x1 >> (32 - r))
        x1 = x1 ^ x0
    return x0, x1


def _threefry2x32(x0, x1):
    x0 = x0 + _KS0
    x1 = x1 + _KS1
    x0, x1 = _rounds(x0, x1, _ROT0)
    x0 = x0 + _KS1
    x1 = x1 + np.uint32(_KS2 + np.uint32(1))
    x0, x1 = _rounds(x0, x1, _ROT1)
    x0 = x0 + _KS2
    x1 = x1 + np.uint32(_KS0 + np.uint32(2))
    x0, x1 = _rounds(x0, x1, _ROT0)
    x0 = x0 + _KS0
    x1 = x1 + np.uint32(_KS1 + np.uint32(3))
    x0, x1 = _rounds(x0, x1, _ROT1)
    x0 = x0 + _KS1
    x1 = x1 + np.uint32(_KS2 + np.uint32(4))
    x0, x1 = _rounds(x0, x1, _ROT0)
    x0 = x0 + _KS2
    x1 = x1 + np.uint32(_KS0 + np.uint32(5))
    return x0, x1


_TINY = np.float32(np.finfo(np.float32).tiny)
_SPAN = np.float32(np.float32(1.0) - _TINY)


def _bits_to_gumbel(bits):
    fb = (bits >> np.uint32(9)) | np.uint32(0x3F800000)
    f = lax.bitcast_convert_type(fb, jnp.float32) - np.float32(1.0)
    u = jnp.maximum(_TINY, f * _SPAN + _TINY)
    return -jnp.log(-jnp.log(u))


def _sample_kernel(x_ref, o_ref, run_ref, idx_ref):
    k = pl.program_id(0)

    @pl.when(k == 0)
    def _init():
        run_ref[...] = jnp.full((ROWS, BC), -jnp.inf, jnp.float32)
        idx_ref[...] = jnp.zeros((ROWS, BC), jnp.int32)

    base = k * BC
    for j in range(NCHUNK):
        sl = slice(j * CW, (j + 1) * CW)
        rows = lax.broadcasted_iota(jnp.int32, (ROWS, CW), 0)
        cols = lax.broadcasted_iota(jnp.int32, (ROWS, CW), 1) + (base + j * CW)
        flat = (rows * COLS + cols).astype(jnp.uint32)
        o0, o1 = _threefry2x32(np.uint32(0), flat)
        g = _bits_to_gumbel(o0 ^ o1)
        val = jnp.where(cols < COLS, x_ref[:, sl] + g, -jnp.inf)

        run = run_ref[:, sl]
        m = val > run
        run_ref[:, sl] = jnp.where(m, val, run)
        idx_ref[:, sl] = jnp.where(m, cols, idx_ref[:, sl])

    @pl.when(k == GRID - 1)
    def _finalize():
        run = run_ref[...]
        idx = idx_ref[...]
        m = jnp.max(run, axis=1, keepdims=True)
        cand = jnp.where(run == m, idx, jnp.int32(np.iinfo(np.int32).max))
        o_ref[...] = jnp.min(cand, axis=1, keepdims=True)


def kernel(x):
    out = pl.pallas_call(
        _sample_kernel,
        grid=(GRID,),
        in_specs=[pl.BlockSpec((ROWS, BC), lambda k: (0, k))],
        out_specs=pl.BlockSpec((ROWS, 1), lambda k: (0, 0)),
        out_shape=jax.ShapeDtypeStruct((ROWS, 1), jnp.int32),
        scratch_shapes=[
            pltpu.VMEM((ROWS, BC), jnp.float32),
            pltpu.VMEM((ROWS, BC), jnp.int32),
        ],
    )(x)
    return out[:, 0]
